# initial kernel scaffold (unmeasured)
import jax
import jax.numpy as jnp
from jax import lax
from jax.experimental import pallas as pl
from jax.experimental.pallas import tpu as pltpu

N_DEV = 4
M_LOC = 1024
K_LOC = 1024
M_GLOB = 4096
K_GLOB = 4096
N_GLOB = 8192
NT = 512
N_STEPS = N_GLOB // NT

_GELU_C = 0.7978845608028654


def _gelu(y):
    return 0.5 * y * (1.0 + jnp.tanh(_GELU_C * (y + 0.044715 * y * y * y)))


def kernel(x, w_mat):
    def body(x_ref, w_ref, out_ref, xb_ref, xt_ref, send_sems, recv_sems):
        my = lax.axis_index("i")
        step = pl.program_id(0)

        @pl.when(step == 0)
        def _comm():
            bsem = pltpu.get_barrier_semaphore()
            for d in range(1, N_DEV):
                pl.semaphore_signal(
                    bsem, inc=1,
                    device_id=(lax.rem(my + d, N_DEV),),
                    device_id_type=pl.DeviceIdType.MESH,
                )
            pl.semaphore_wait(bsem, N_DEV - 1)

            for s in range(N_DEV):
                xb_ref[s] = x_ref[s * M_LOC:(s + 1) * M_LOC, :].astype(
                    jnp.bfloat16
                )
            xt_ref[pl.ds(my, 1)] = xb_ref[pl.ds(my, 1)]

            rdmas = []
            for d in range(1, N_DEV):
                tgt = lax.rem(my + d, N_DEV)
                r = pltpu.make_async_remote_copy(
                    src_ref=xb_ref.at[tgt],
                    dst_ref=xt_ref.at[my],
                    send_sem=send_sems.at[d - 1],
                    recv_sem=recv_sems.at[d - 1],
                    device_id=(tgt,),
                    device_id_type=pl.DeviceIdType.MESH,
                )
                r.start()
                rdmas.append(r)
            for r in rdmas:
                r.wait()

        w_bf = w_ref[...].astype(jnp.bfloat16)
        acc = jnp.zeros((M_LOC, NT), jnp.float32)
        for s in range(N_DEV):
            acc = acc + lax.dot_general(
                xt_ref[s],
                w_bf[s * K_LOC:(s + 1) * K_LOC, :],
                (((1,), (0,)), ((), ())),
                preferred_element_type=jnp.float32,
            )
        out_ref[...] = _gelu(acc)

    return pl.pallas_call(
        body,
        grid=(N_STEPS,),
        in_specs=[
            pl.BlockSpec((M_GLOB, K_LOC), lambda n: (0, 0)),
            pl.BlockSpec((K_GLOB, NT), lambda n: (0, n)),
        ],
        out_specs=pl.BlockSpec((M_LOC, NT), lambda n: (0, n)),
        out_shape=jax.ShapeDtypeStruct((M_LOC, N_GLOB), jnp.float32),
        scratch_shapes=[
            pltpu.VMEM((N_DEV, M_LOC, K_LOC), jnp.bfloat16),
            pltpu.VMEM((N_DEV, M_LOC, K_LOC), jnp.bfloat16),
            pltpu.SemaphoreType.DMA((N_DEV - 1,)),
            pltpu.SemaphoreType.DMA((N_DEV - 1,)),
        ],
        compiler_params=pltpu.CompilerParams(collective_id=0),
    )(x, w_mat)


# baseline (device time: 165089 ns/iter reference)
import jax
import jax.numpy as jnp
from jax import lax
from jax.experimental import pallas as pl
from jax.experimental.pallas import tpu as pltpu

N_DEV = 4
M_LOC = 1024
K_LOC = 1024
M_GLOB = 4096
K_GLOB = 4096
N_GLOB = 8192
NT = 512
N_STEPS = N_GLOB // NT

_GELU_C = 0.7978845608028654


def _gelu(y):
    return 0.5 * y * (1.0 + jnp.tanh(_GELU_C * (y + 0.044715 * y * y * y)))


def kernel(x, w_mat):
    def body(x_ref, w_ref, out_ref, xb_ref, xt_ref, send_sems, recv_sems):
        my = lax.axis_index("i")
        step = pl.program_id(0)

        @pl.when(step == 0)
        def _comm():
            bsem = pltpu.get_barrier_semaphore()
            for d in range(1, N_DEV):
                pl.semaphore_signal(
                    bsem, inc=1,
                    device_id=(lax.rem(my + d, N_DEV),),
                    device_id_type=pl.DeviceIdType.MESH,
                )
            pl.semaphore_wait(bsem, N_DEV - 1)

            for s in range(N_DEV):
                xb_ref[s] = x_ref[s * M_LOC:(s + 1) * M_LOC, :].astype(
                    jnp.bfloat16
                )
            xt_ref[pl.ds(my, 1)] = xb_ref[pl.ds(my, 1)]

            rdmas = []
            for d in range(1, N_DEV):
                tgt = lax.rem(my + d, N_DEV)
                r = pltpu.make_async_remote_copy(
                    src_ref=xb_ref.at[tgt],
                    dst_ref=xt_ref.at[my],
                    send_sem=send_sems.at[d - 1],
                    recv_sem=recv_sems.at[d - 1],
                    device_id=(tgt,),
                    device_id_type=pl.DeviceIdType.MESH,
                )
                r.start()
                rdmas.append(r)
            for r in rdmas:
                r.wait()

        w_bf = w_ref[...].astype(jnp.bfloat16)
        acc = jnp.zeros((M_LOC, NT), jnp.float32)
        for s in range(N_DEV):
            acc = acc + lax.dot_general(
                xt_ref[s],
                w_bf[s * K_LOC:(s + 1) * K_LOC, :],
                (((1,), (0,)), ((), ())),
                preferred_element_type=jnp.float32,
            )
        out_ref[...] = _gelu(acc)

    return pl.pallas_call(
        body,
        grid=(N_STEPS,),
        in_specs=[
            pl.BlockSpec((M_GLOB, K_LOC), lambda n: (0, 0)),
            pl.BlockSpec((K_GLOB, NT), lambda n: (0, n)),
        ],
        out_specs=pl.BlockSpec((M_LOC, NT), lambda n: (0, n)),
        out_shape=jax.ShapeDtypeStruct((M_LOC, N_GLOB), jnp.float32),
        scratch_shapes=[
            pltpu.VMEM((N_DEV, M_LOC, K_LOC), jnp.bfloat16),
            pltpu.VMEM((N_DEV, M_LOC, K_LOC), jnp.bfloat16),
            pltpu.SemaphoreType.DMA((N_DEV - 1,)),
            pltpu.SemaphoreType.DMA((N_DEV - 1,)),
        ],
        compiler_params=pltpu.CompilerParams(
            collective_id=0, vmem_limit_bytes=64 * 1024 * 1024
        ),
    )(x, w_mat)


# device time: 158646 ns/iter; 1.0406x vs baseline; 1.0406x over previous
import jax
import jax.numpy as jnp
from jax import lax
from jax.experimental import pallas as pl
from jax.experimental.pallas import tpu as pltpu

N_DEV = 4
M_LOC = 1024
K_LOC = 1024
N_GLOB = 8192
NT = 512
N_STEPS = N_GLOB // NT

_PHASE_ORDER = (0, 1, 3, 2)

_GELU_C = 0.7978845608028654


def _gelu(y):
    return 0.5 * y * (1.0 + jnp.tanh(_GELU_C * (y + 0.044715 * y * y * y)))


def kernel(x, w_mat):
    def body(x_ref, w_ref, out_ref, xt_ref, wstg_ref, send_sems, recv_sems,
             wsems):
        my = lax.axis_index("i")

        bsem = pltpu.get_barrier_semaphore()
        for d in range(1, N_DEV):
            pl.semaphore_signal(
                bsem, inc=1,
                device_id=(lax.rem(my + d, N_DEV),),
                device_id_type=pl.DeviceIdType.MESH,
            )
        pl.semaphore_wait(bsem, N_DEV - 1)

        rdmas = {}
        for dl in (1, 3, 2):
            tgt = lax.rem(my + dl, N_DEV)
            r = pltpu.make_async_remote_copy(
                src_ref=x_ref.at[pl.ds(tgt * M_LOC, M_LOC), :],
                dst_ref=xt_ref.at[4 - dl],
                send_sem=send_sems.at[dl - 1],
                recv_sem=recv_sems.at[dl - 1],
                device_id=(tgt,),
                device_id_type=pl.DeviceIdType.MESH,
            )
            r.start()
            rdmas[dl] = r

        def w_dma(t):
            pi, n = divmod(t, N_STEPS)
            slot = lax.rem(my + _PHASE_ORDER[pi], N_DEV)
            return pltpu.make_async_copy(
                w_ref.at[pl.ds(slot * K_LOC, K_LOC), pl.ds(n * NT, NT)],
                wstg_ref.at[t % 2],
                wsems.at[t % 2],
            )

        n_tiles = N_DEV * N_STEPS
        dmas = {t: w_dma(t) for t in range(n_tiles)}
        dmas[0].start()
        dmas[1].start()

        for t in range(n_tiles):
            pi, n = divmod(t, N_STEPS)
            d = _PHASE_ORDER[pi]
            if n == 0 and d != 0:
                rdmas[4 - d].wait_recv()
            dmas[t].wait()
            wv = wstg_ref[t % 2].astype(jnp.bfloat16)
            if d == 0:
                xv = x_ref[pl.ds(my * M_LOC, M_LOC), :]
            else:
                xv = xt_ref[d]
            contrib = lax.dot_general(
                xv, wv, (((1,), (0,)), ((), ())),
                preferred_element_type=jnp.float32,
            )
            col = pl.ds(n * NT, NT)
            if pi == 0:
                out_ref[:, col] = contrib
            elif pi < N_DEV - 1:
                out_ref[:, col] = out_ref[:, col] + contrib
            else:
                out_ref[:, col] = _gelu(out_ref[:, col] + contrib)
            if t + 2 < n_tiles:
                dmas[t + 2].start()

        for dl in (1, 2, 3):
            rdmas[dl].wait_send()

    x_bf = x.astype(jnp.bfloat16)
    return pl.pallas_call(
        body,
        out_shape=jax.ShapeDtypeStruct((M_LOC, N_GLOB), jnp.float32),
        in_specs=[
            pl.BlockSpec(memory_space=pltpu.MemorySpace.VMEM),
            pl.BlockSpec(memory_space=pl.ANY),
        ],
        out_specs=pl.BlockSpec(memory_space=pltpu.MemorySpace.VMEM),
        scratch_shapes=[
            pltpu.VMEM((N_DEV, M_LOC, K_LOC), jnp.bfloat16),
            pltpu.VMEM((2, K_LOC, NT), jnp.float32),
            pltpu.SemaphoreType.DMA((N_DEV - 1,)),
            pltpu.SemaphoreType.DMA((N_DEV - 1,)),
            pltpu.SemaphoreType.DMA((2,)),
        ],
        compiler_params=pltpu.CompilerParams(
            collective_id=0, vmem_limit_bytes=64 * 1024 * 1024
        ),
    )(x_bf, w_mat)


# device time: 140687 ns/iter; 1.1734x vs baseline; 1.1277x over previous
import jax
import jax.numpy as jnp
from jax import lax
from jax.experimental import pallas as pl
from jax.experimental.pallas import tpu as pltpu

N_DEV = 4
M_LOC = 1024
K_LOC = 1024
N_GLOB = 8192
NT = 1024
N_STEPS = N_GLOB // NT
N_BUF = 3

_PHASE_ORDER = (0, 1, 3, 2)

_GELU_C = 0.7978845608028654


def _gelu(y):
    return 0.5 * y * (1.0 + jnp.tanh(_GELU_C * (y + 0.044715 * y * y * y)))


def kernel(x, w_mat):
    def body(x_ref, w_ref, out_ref, xt_ref, wstg_ref, send_sems, recv_sems,
             wsems, xsem):
        my = lax.axis_index("i")

        bsem = pltpu.get_barrier_semaphore()
        for d in range(1, N_DEV):
            pl.semaphore_signal(
                bsem, inc=1,
                device_id=(lax.rem(my + d, N_DEV),),
                device_id_type=pl.DeviceIdType.MESH,
            )
        pl.semaphore_wait(bsem, N_DEV - 1)

        rdmas = {}
        for dl in (1, 3, 2):
            tgt = lax.rem(my + dl, N_DEV)
            r = pltpu.make_async_remote_copy(
                src_ref=x_ref.at[pl.ds(tgt * M_LOC, M_LOC), :],
                dst_ref=xt_ref.at[4 - dl],
                send_sem=send_sems.at[dl - 1],
                recv_sem=recv_sems.at[dl - 1],
                device_id=(tgt,),
                device_id_type=pl.DeviceIdType.MESH,
            )
            r.start()
            rdmas[dl] = r

        xcopy = pltpu.make_async_copy(
            x_ref.at[pl.ds(my * M_LOC, M_LOC), :], xt_ref.at[0], xsem
        )
        xcopy.start()

        def w_dma(t):
            pi, n = divmod(t, N_STEPS)
            slot = lax.rem(my + _PHASE_ORDER[pi], N_DEV)
            return pltpu.make_async_copy(
                w_ref.at[pl.ds(slot * K_LOC, K_LOC), pl.ds(n * NT, NT)],
                wstg_ref.at[t % N_BUF],
                wsems.at[t % N_BUF],
            )

        n_tiles = N_DEV * N_STEPS
        dmas = {t: w_dma(t) for t in range(n_tiles)}
        for t in range(N_BUF - 1):
            dmas[t].start()

        for t in range(n_tiles):
            pi, n = divmod(t, N_STEPS)
            d = _PHASE_ORDER[pi]
            if t == 0:
                xcopy.wait()
            if n == 0 and d != 0:
                rdmas[4 - d].wait_recv()
            dmas[t].wait()
            wv = wstg_ref[t % N_BUF].astype(jnp.bfloat16)
            xv = xt_ref[0] if d == 0 else xt_ref[d]
            contrib = lax.dot_general(
                xv, wv, (((1,), (0,)), ((), ())),
                preferred_element_type=jnp.float32,
            )
            col = pl.ds(n * NT, NT)
            if pi == 0:
                out_ref[:, col] = contrib
            elif pi < N_DEV - 1:
                out_ref[:, col] = out_ref[:, col] + contrib
            else:
                out_ref[:, col] = _gelu(out_ref[:, col] + contrib)
            if t + N_BUF - 1 < n_tiles:
                dmas[t + N_BUF - 1].start()

        for dl in (1, 2, 3):
            rdmas[dl].wait_send()

    x_bf = x.astype(jnp.bfloat16)
    return pl.pallas_call(
        body,
        out_shape=jax.ShapeDtypeStruct((M_LOC, N_GLOB), jnp.float32),
        in_specs=[
            pl.BlockSpec(memory_space=pl.ANY),
            pl.BlockSpec(memory_space=pl.ANY),
        ],
        out_specs=pl.BlockSpec(memory_space=pltpu.MemorySpace.VMEM),
        scratch_shapes=[
            pltpu.VMEM((N_DEV, M_LOC, K_LOC), jnp.bfloat16),
            pltpu.VMEM((N_BUF, K_LOC, NT), jnp.float32),
            pltpu.SemaphoreType.DMA((N_DEV - 1,)),
            pltpu.SemaphoreType.DMA((N_DEV - 1,)),
            pltpu.SemaphoreType.DMA((N_BUF,)),
            pltpu.SemaphoreType.DMA,
        ],
        compiler_params=pltpu.CompilerParams(
            collective_id=0, vmem_limit_bytes=64 * 1024 * 1024
        ),
    )(x_bf, w_mat)
